# baseline (device time: 307711 ns/iter reference)
import jax
import jax.numpy as jnp
from jax import lax
from jax.experimental import pallas as pl
from jax.experimental.pallas import tpu as pltpu


def kernel(Q, K, V):
    b, q, h, d = Q.shape
    kloc = K.shape[1]
    hd = h * d
    scale = d ** -0.5

    Q3 = Q.reshape(b, q, hd)
    K3 = K.reshape(b, kloc, hd)
    V3 = V.reshape(b, kloc, hd)

    n_chunks = 4
    kc = kloc // n_chunks

    def partial_body(q_ref, k_ref, v_ref, n_ref, d_ref):
        c = pl.program_id(1)
        for hh in range(h):
            sl = slice(hh * d, (hh + 1) * d)
            qh = q_ref[0][:, sl].astype(jnp.bfloat16)
            kh = k_ref[0][:, sl].astype(jnp.bfloat16)
            vh = v_ref[0][:, sl].astype(jnp.bfloat16)
            s = lax.dot_general(
                qh, kh, (((1,), (1,)), ((), ())),
                preferred_element_type=jnp.float32) * scale
            p = jnp.exp(s)
            n = lax.dot_general(
                p.astype(jnp.bfloat16), vh, (((1,), (0,)), ((), ())),
                preferred_element_type=jnp.float32)
            l = jnp.broadcast_to(
                jnp.sum(p, axis=1, keepdims=True), (q, d))

            @pl.when(c == 0)
            def _():
                n_ref[0, :, sl] = n
                d_ref[0, :, sl] = l

            @pl.when(c != 0)
            def _():
                n_ref[0, :, sl] += n
                d_ref[0, :, sl] += l

    N3, D3 = pl.pallas_call(
        partial_body,
        grid=(b, n_chunks),
        in_specs=[
            pl.BlockSpec((1, q, hd), lambda i, c: (i, 0, 0)),
            pl.BlockSpec((1, kc, hd), lambda i, c: (i, c, 0)),
            pl.BlockSpec((1, kc, hd), lambda i, c: (i, c, 0)),
        ],
        out_specs=[
            pl.BlockSpec((1, q, hd), lambda i, c: (i, 0, 0)),
            pl.BlockSpec((1, q, hd), lambda i, c: (i, 0, 0)),
        ],
        out_shape=[
            jax.ShapeDtypeStruct((b, q, hd), jnp.float32),
            jax.ShapeDtypeStruct((b, q, hd), jnp.float32),
        ],
    )(Q3, K3, V3)

    def reduce_body(n_ref, d_ref, o_ref, ncom_ref, dcom_ref,
                    send_sem, recv_sem):
        my_x = lax.axis_index("x")
        my_y = lax.axis_index("y")
        my_z = lax.axis_index("z")
        nbr = (1 - my_x, my_y, my_z)
        copy_n = pltpu.make_async_remote_copy(
            src_ref=n_ref, dst_ref=ncom_ref,
            send_sem=send_sem.at[0], recv_sem=recv_sem.at[0],
            device_id=nbr, device_id_type=pl.DeviceIdType.MESH)
        copy_d = pltpu.make_async_remote_copy(
            src_ref=d_ref, dst_ref=dcom_ref,
            send_sem=send_sem.at[1], recv_sem=recv_sem.at[1],
            device_id=nbr, device_id_type=pl.DeviceIdType.MESH)
        copy_n.start()
        copy_d.start()
        copy_n.wait()
        copy_d.wait()
        o_ref[...] = (n_ref[...] + ncom_ref[...]) / (d_ref[...] + dcom_ref[...])

    O3 = pl.pallas_call(
        reduce_body,
        in_specs=[pl.BlockSpec(memory_space=pltpu.VMEM)] * 2,
        out_specs=pl.BlockSpec(memory_space=pltpu.VMEM),
        out_shape=jax.ShapeDtypeStruct((b, q, hd), jnp.float32),
        scratch_shapes=[
            pltpu.VMEM((b, q, hd), jnp.float32),
            pltpu.VMEM((b, q, hd), jnp.float32),
            pltpu.SemaphoreType.DMA((2,)),
            pltpu.SemaphoreType.DMA((2,)),
        ],
    )(N3, D3)

    return O3.reshape(b, q, h, d)


# device time: 190989 ns/iter; 1.6111x vs baseline; 1.6111x over previous
import jax
import jax.numpy as jnp
from jax import lax
from jax.experimental import pallas as pl
from jax.experimental.pallas import tpu as pltpu


def kernel(Q, K, V):
    b, q, h, d = Q.shape
    kloc = K.shape[1]
    scale = d ** -0.5

    nc = 4
    kc = kloc // nc

    def partial_body(q_ref, k_ref, v_ref, n_ref, d_ref):
        c = pl.program_id(1)
        qb = q_ref[0].astype(jnp.bfloat16)
        kb = k_ref[0].astype(jnp.bfloat16)
        vb = v_ref[0].astype(jnp.bfloat16)
        s = lax.dot_general(
            qb, kb, (((2,), (2,)), ((1,), (1,))),
            preferred_element_type=jnp.float32) * scale
        p = jnp.exp(s)
        n = lax.dot_general(
            p.astype(jnp.bfloat16), vb, (((2,), (0,)), ((0,), (1,))),
            preferred_element_type=jnp.float32)
        l = jnp.broadcast_to(
            jnp.sum(p, axis=2, keepdims=True), (h, q, d))

        @pl.when(c == 0)
        def _():
            n_ref[0] = n
            d_ref[0] = l

        @pl.when(c != 0)
        def _():
            n_ref[0] += n
            d_ref[0] += l

    N4, D4 = pl.pallas_call(
        partial_body,
        grid=(b, nc),
        in_specs=[
            pl.BlockSpec((1, q, h, d), lambda i, c: (i, 0, 0, 0)),
            pl.BlockSpec((1, kc, h, d), lambda i, c: (i, c, 0, 0)),
            pl.BlockSpec((1, kc, h, d), lambda i, c: (i, c, 0, 0)),
        ],
        out_specs=[
            pl.BlockSpec((1, h, q, d), lambda i, c: (i, 0, 0, 0)),
            pl.BlockSpec((1, h, q, d), lambda i, c: (i, 0, 0, 0)),
        ],
        out_shape=[
            jax.ShapeDtypeStruct((b, h, q, d), jnp.float32),
            jax.ShapeDtypeStruct((b, h, q, d), jnp.float32),
        ],
    )(Q, K, V)

    def reduce_body(n_ref, d_ref, o_ref, ncom_ref, dcom_ref,
                    send_sem, recv_sem):
        bi = pl.program_id(0)

        @pl.when(bi == 0)
        def _():
            my_x = lax.axis_index("x")
            my_y = lax.axis_index("y")
            my_z = lax.axis_index("z")
            nbr = (1 - my_x, my_y, my_z)
            copy_n = pltpu.make_async_remote_copy(
                src_ref=n_ref, dst_ref=ncom_ref,
                send_sem=send_sem.at[0], recv_sem=recv_sem.at[0],
                device_id=nbr, device_id_type=pl.DeviceIdType.MESH)
            copy_d = pltpu.make_async_remote_copy(
                src_ref=d_ref, dst_ref=dcom_ref,
                send_sem=send_sem.at[1], recv_sem=recv_sem.at[1],
                device_id=nbr, device_id_type=pl.DeviceIdType.MESH)
            copy_n.start()
            copy_d.start()
            copy_n.wait()
            copy_d.wait()

        nsum = n_ref[bi] + ncom_ref[bi]
        dsum = d_ref[bi] + dcom_ref[bi]
        o = nsum / dsum
        for hh in range(h):
            o_ref[0, :, hh, :] = o[hh]

    return pl.pallas_call(
        reduce_body,
        grid=(b,),
        in_specs=[pl.BlockSpec(memory_space=pltpu.VMEM)] * 2,
        out_specs=pl.BlockSpec((1, q, h, d), lambda i: (i, 0, 0, 0)),
        out_shape=jax.ShapeDtypeStruct((b, q, h, d), jnp.float32),
        scratch_shapes=[
            pltpu.VMEM((b, h, q, d), jnp.float32),
            pltpu.VMEM((b, h, q, d), jnp.float32),
            pltpu.SemaphoreType.DMA((2,)),
            pltpu.SemaphoreType.DMA((2,)),
        ],
    )(N4, D4)


# device time: 59207 ns/iter; 5.1972x vs baseline; 3.2258x over previous
import jax
import jax.numpy as jnp
from jax import lax
from jax.experimental import pallas as pl
from jax.experimental.pallas import tpu as pltpu


def kernel(Q, K, V):
    b, q, h, d = Q.shape
    kloc = K.shape[1]
    scale = d ** -0.5

    bq = b // 4
    t = 2 * lax.axis_index("y") + lax.axis_index("z")
    t_arr = jnp.reshape(t.astype(jnp.int32), (1,))

    nc = 4
    kc = kloc // nc

    def partial_body(t_ref, q_ref, k_ref, v_ref, n_ref, d_ref):
        c = pl.program_id(1)
        qb = q_ref[0].astype(jnp.bfloat16)
        kb = k_ref[0].astype(jnp.bfloat16)
        vb = v_ref[0].astype(jnp.bfloat16)
        s = lax.dot_general(
            qb, kb, (((2,), (2,)), ((1,), (1,))),
            preferred_element_type=jnp.float32) * scale
        p = jnp.exp(s)
        n = lax.dot_general(
            p.astype(jnp.bfloat16), vb, (((2,), (0,)), ((0,), (1,))),
            preferred_element_type=jnp.float32)
        l = jnp.broadcast_to(
            jnp.sum(p, axis=2, keepdims=True), (h, q, d))

        @pl.when(c == 0)
        def _():
            n_ref[0] = n
            d_ref[0] = l

        @pl.when(c != 0)
        def _():
            n_ref[0] += n
            d_ref[0] += l

    grid_spec = pltpu.PrefetchScalarGridSpec(
        num_scalar_prefetch=1,
        grid=(bq, nc),
        in_specs=[
            pl.BlockSpec((1, q, h, d), lambda i, c, tr: (tr[0] * bq + i, 0, 0, 0)),
            pl.BlockSpec((1, kc, h, d), lambda i, c, tr: (tr[0] * bq + i, c, 0, 0)),
            pl.BlockSpec((1, kc, h, d), lambda i, c, tr: (tr[0] * bq + i, c, 0, 0)),
        ],
        out_specs=[
            pl.BlockSpec((1, h, q, d), lambda i, c, tr: (i, 0, 0, 0)),
            pl.BlockSpec((1, h, q, d), lambda i, c, tr: (i, 0, 0, 0)),
        ],
    )

    Nq, Dq = pl.pallas_call(
        partial_body,
        grid_spec=grid_spec,
        out_shape=[
            jax.ShapeDtypeStruct((bq, h, q, d), jnp.float32),
            jax.ShapeDtypeStruct((bq, h, q, d), jnp.float32),
        ],
    )(t_arr, Q, K, V)

    def reduce_body(n_ref, d_ref, o_ref, ncom_ref, dcom_ref,
                    xsend_sem, xrecv_sem, asend_sem, arecv_sem):
        my_x = lax.axis_index("x")
        my_y = lax.axis_index("y")
        my_z = lax.axis_index("z")
        my_t = 2 * my_y + my_z
        xpeer = (1 - my_x, my_y, my_z)
        gpeers = [
            (my_x, 1 - my_y, my_z),
            (my_x, my_y, 1 - my_z),
            (my_x, 1 - my_y, 1 - my_z),
        ]

        barrier_sem = pltpu.get_barrier_semaphore()
        for peer in [xpeer] + gpeers:
            pl.semaphore_signal(barrier_sem, inc=1, device_id=peer,
                                device_id_type=pl.DeviceIdType.MESH)
        pl.semaphore_wait(barrier_sem, 4)

        copy_n = pltpu.make_async_remote_copy(
            src_ref=n_ref, dst_ref=ncom_ref,
            send_sem=xsend_sem.at[0], recv_sem=xrecv_sem.at[0],
            device_id=xpeer, device_id_type=pl.DeviceIdType.MESH)
        copy_d = pltpu.make_async_remote_copy(
            src_ref=d_ref, dst_ref=dcom_ref,
            send_sem=xsend_sem.at[1], recv_sem=xrecv_sem.at[1],
            device_id=xpeer, device_id_type=pl.DeviceIdType.MESH)
        copy_n.start()
        copy_d.start()
        copy_n.wait()
        copy_d.wait()

        osum = (n_ref[...] + ncom_ref[...]) / (d_ref[...] + dcom_ref[...])
        rows = pl.ds(my_t * bq, bq)
        for hh in range(h):
            o_ref[rows, :, hh, :] = osum[:, hh, :, :]

        copies = []
        for r, peer in enumerate(gpeers):
            cp = pltpu.make_async_remote_copy(
                src_ref=o_ref.at[rows],
                dst_ref=o_ref.at[rows],
                send_sem=asend_sem.at[r], recv_sem=arecv_sem.at[r],
                device_id=peer, device_id_type=pl.DeviceIdType.MESH)
            cp.start()
            copies.append(cp)
        for cp in copies:
            cp.wait()

    return pl.pallas_call(
        reduce_body,
        in_specs=[pl.BlockSpec(memory_space=pltpu.VMEM)] * 2,
        out_specs=pl.BlockSpec(memory_space=pltpu.VMEM),
        out_shape=jax.ShapeDtypeStruct((b, q, h, d), jnp.float32),
        scratch_shapes=[
            pltpu.VMEM((bq, h, q, d), jnp.float32),
            pltpu.VMEM((bq, h, q, d), jnp.float32),
            pltpu.SemaphoreType.DMA((2,)),
            pltpu.SemaphoreType.DMA((2,)),
            pltpu.SemaphoreType.DMA((3,)),
            pltpu.SemaphoreType.DMA((3,)),
        ],
        compiler_params=pltpu.CompilerParams(collective_id=0),
    )(Nq, Dq)


# device time: 30433 ns/iter; 10.1111x vs baseline; 1.9455x over previous
import jax
import jax.numpy as jnp
from jax import lax
from jax.experimental import pallas as pl
from jax.experimental.pallas import tpu as pltpu


def kernel(Q, K, V):
    b, q, h, d = Q.shape
    kloc = K.shape[1]
    scale = d ** -0.5

    bq = b // 4
    t = 2 * lax.axis_index("y") + lax.axis_index("z")
    t_arr = jnp.reshape(t.astype(jnp.int32), (1,))

    nc = 4
    kc = kloc // nc
    rows = q * h
    cols = kc * h

    def partial_body(t_ref, q_ref, k_ref, v_ref, n_ref, d_ref, bias_ref):
        i = pl.program_id(0)
        c = pl.program_id(1)

        @pl.when(jnp.logical_and(i == 0, c == 0))
        def _():
            rh = lax.broadcasted_iota(jnp.int32, (rows, cols), 0) % h
            ch = lax.broadcasted_iota(jnp.int32, (rows, cols), 1) % h
            bias_ref[...] = jnp.where(rh == ch, 0.0, -1e30).astype(jnp.float32)

        q2 = q_ref[0].reshape(rows, d).astype(jnp.bfloat16)
        k2 = k_ref[0].reshape(cols, d).astype(jnp.bfloat16)
        v2 = v_ref[0].reshape(cols, d).astype(jnp.bfloat16)
        s = lax.dot_general(
            q2, k2, (((1,), (1,)), ((), ())),
            preferred_element_type=jnp.float32)
        p = jnp.exp(s * scale + bias_ref[...])
        n = lax.dot_general(
            p.astype(jnp.bfloat16), v2, (((1,), (0,)), ((), ())),
            preferred_element_type=jnp.float32)
        l = jnp.broadcast_to(
            jnp.sum(p, axis=1, keepdims=True), (rows, d))

        @pl.when(c == 0)
        def _():
            n_ref[0] = n.reshape(q, h, d)
            d_ref[0] = l.reshape(q, h, d)

        @pl.when(c != 0)
        def _():
            n_ref[0] += n.reshape(q, h, d)
            d_ref[0] += l.reshape(q, h, d)

    grid_spec = pltpu.PrefetchScalarGridSpec(
        num_scalar_prefetch=1,
        grid=(bq, nc),
        in_specs=[
            pl.BlockSpec((1, q, h, d), lambda i, c, tr: (tr[0] * bq + i, 0, 0, 0)),
            pl.BlockSpec((1, kc, h, d), lambda i, c, tr: (tr[0] * bq + i, c, 0, 0)),
            pl.BlockSpec((1, kc, h, d), lambda i, c, tr: (tr[0] * bq + i, c, 0, 0)),
        ],
        out_specs=[
            pl.BlockSpec((1, q, h, d), lambda i, c, tr: (i, 0, 0, 0)),
            pl.BlockSpec((1, q, h, d), lambda i, c, tr: (i, 0, 0, 0)),
        ],
        scratch_shapes=[
            pltpu.VMEM((rows, cols), jnp.float32),
        ],
    )

    Nq, Dq = pl.pallas_call(
        partial_body,
        grid_spec=grid_spec,
        out_shape=[
            jax.ShapeDtypeStruct((bq, q, h, d), jnp.float32),
            jax.ShapeDtypeStruct((bq, q, h, d), jnp.float32),
        ],
    )(t_arr, Q, K, V)

    def reduce_body(n_ref, d_ref, o_ref, ncom_ref, dcom_ref,
                    xsend_sem, xrecv_sem, asend_sem, arecv_sem):
        my_x = lax.axis_index("x")
        my_y = lax.axis_index("y")
        my_z = lax.axis_index("z")
        my_t = 2 * my_y + my_z
        xpeer = (1 - my_x, my_y, my_z)
        gpeers = [
            (my_x, 1 - my_y, my_z),
            (my_x, my_y, 1 - my_z),
            (my_x, 1 - my_y, 1 - my_z),
        ]

        barrier_sem = pltpu.get_barrier_semaphore()
        for peer in [xpeer] + gpeers:
            pl.semaphore_signal(barrier_sem, inc=1, device_id=peer,
                                device_id_type=pl.DeviceIdType.MESH)
        pl.semaphore_wait(barrier_sem, 4)

        copy_n = pltpu.make_async_remote_copy(
            src_ref=n_ref, dst_ref=ncom_ref,
            send_sem=xsend_sem.at[0], recv_sem=xrecv_sem.at[0],
            device_id=xpeer, device_id_type=pl.DeviceIdType.MESH)
        copy_d = pltpu.make_async_remote_copy(
            src_ref=d_ref, dst_ref=dcom_ref,
            send_sem=xsend_sem.at[1], recv_sem=xrecv_sem.at[1],
            device_id=xpeer, device_id_type=pl.DeviceIdType.MESH)
        copy_n.start()
        copy_d.start()
        copy_n.wait()
        copy_d.wait()

        my_rows = pl.ds(my_t * bq, bq)
        o_ref[my_rows] = (
            (n_ref[...] + ncom_ref[...]) / (d_ref[...] + dcom_ref[...]))

        copies = []
        for r, peer in enumerate(gpeers):
            cp = pltpu.make_async_remote_copy(
                src_ref=o_ref.at[my_rows],
                dst_ref=o_ref.at[my_rows],
                send_sem=asend_sem.at[r], recv_sem=arecv_sem.at[r],
                device_id=peer, device_id_type=pl.DeviceIdType.MESH)
            cp.start()
            copies.append(cp)
        for cp in copies:
            cp.wait()

    return pl.pallas_call(
        reduce_body,
        in_specs=[pl.BlockSpec(memory_space=pltpu.VMEM)] * 2,
        out_specs=pl.BlockSpec(memory_space=pltpu.VMEM),
        out_shape=jax.ShapeDtypeStruct((b, q, h, d), jnp.float32),
        scratch_shapes=[
            pltpu.VMEM((bq, q, h, d), jnp.float32),
            pltpu.VMEM((bq, q, h, d), jnp.float32),
            pltpu.SemaphoreType.DMA((2,)),
            pltpu.SemaphoreType.DMA((2,)),
            pltpu.SemaphoreType.DMA((3,)),
            pltpu.SemaphoreType.DMA((3,)),
        ],
        compiler_params=pltpu.CompilerParams(collective_id=0),
    )(Nq, Dq)


# device time: 27867 ns/iter; 11.0421x vs baseline; 1.0921x over previous
import jax
import jax.numpy as jnp
from jax import lax
from jax.experimental import pallas as pl
from jax.experimental.pallas import tpu as pltpu


def kernel(Q, K, V):
    b, q, h, d = Q.shape
    kloc = K.shape[1]
    scale = d ** -0.5

    bq = b // 4
    t = 2 * lax.axis_index("y") + lax.axis_index("z")
    t_arr = jnp.reshape(t.astype(jnp.int32), (1,))

    nc = 4
    kc = kloc // nc
    rows = q * h
    cols = kc * h

    def body(t_ref, q_ref, k_ref, v_ref, o_ref,
             bias, nacc, dacc, ncom, dcom,
             xsend, xrecv, asend, arecv):
        i = pl.program_id(0)
        c = pl.program_id(1)

        my_x = lax.axis_index("x")
        my_y = lax.axis_index("y")
        my_z = lax.axis_index("z")
        my_t = 2 * my_y + my_z
        xpeer = (1 - my_x, my_y, my_z)
        gpeers = [
            (my_x, 1 - my_y, my_z),
            (my_x, my_y, 1 - my_z),
            (my_x, 1 - my_y, 1 - my_z),
        ]

        def xchg(ii):
            cn = pltpu.make_async_remote_copy(
                src_ref=nacc.at[ii], dst_ref=ncom.at[ii],
                send_sem=xsend.at[0, ii], recv_sem=xrecv.at[0, ii],
                device_id=xpeer, device_id_type=pl.DeviceIdType.MESH)
            cd = pltpu.make_async_remote_copy(
                src_ref=dacc.at[ii], dst_ref=dcom.at[ii],
                send_sem=xsend.at[1, ii], recv_sem=xrecv.at[1, ii],
                device_id=xpeer, device_id_type=pl.DeviceIdType.MESH)
            return cn, cd

        def gath(r, ii):
            row = pl.ds(my_t * bq + ii, 1)
            return pltpu.make_async_remote_copy(
                src_ref=o_ref.at[row], dst_ref=o_ref.at[row],
                send_sem=asend.at[r, ii], recv_sem=arecv.at[r, ii],
                device_id=gpeers[r], device_id_type=pl.DeviceIdType.MESH)

        def combine_and_push(ii):
            cn, cd = xchg(ii)
            cn.wait_recv()
            cd.wait_recv()
            osum = ((nacc[ii] + ncom[ii]) / (dacc[ii] + dcom[ii]))
            o_ref[pl.ds(my_t * bq + ii, 1)] = osum.reshape(1, q, h, d)
            for r in range(3):
                gath(r, ii).start()

        @pl.when(jnp.logical_and(i == 0, c == 0))
        def _():
            rh = lax.broadcasted_iota(jnp.int32, (rows, cols), 0) % h
            ch = lax.broadcasted_iota(jnp.int32, (rows, cols), 1) % h
            bias[...] = jnp.where(rh == ch, 0.0, -1e30).astype(jnp.float32)

            barrier_sem = pltpu.get_barrier_semaphore()
            for peer in [xpeer] + gpeers:
                pl.semaphore_signal(barrier_sem, inc=1, device_id=peer,
                                    device_id_type=pl.DeviceIdType.MESH)
            pl.semaphore_wait(barrier_sem, 4)

        q2 = q_ref[0].reshape(rows, d).astype(jnp.bfloat16)
        k2 = k_ref[0].reshape(cols, d).astype(jnp.bfloat16)
        v2 = v_ref[0].reshape(cols, d).astype(jnp.bfloat16)
        s = lax.dot_general(
            q2, k2, (((1,), (1,)), ((), ())),
            preferred_element_type=jnp.float32)
        p = jnp.exp(s * scale + bias[...])
        n = lax.dot_general(
            p.astype(jnp.bfloat16), v2, (((1,), (0,)), ((), ())),
            preferred_element_type=jnp.float32)
        l = jnp.sum(p, axis=1, keepdims=True)

        @pl.when(c == 0)
        def _():
            nacc[i] = n
            dacc[i] = l

        @pl.when(c != 0)
        def _():
            nacc[i] += n
            dacc[i] += l

        @pl.when(jnp.logical_and(i == 0, c == nc - 1))
        def _():
            cn, cd = xchg(0)
            cn.start()
            cd.start()

        @pl.when(jnp.logical_and(i == 1, c == 1))
        def _():
            combine_and_push(0)

        @pl.when(jnp.logical_and(i == 1, c == nc - 1))
        def _():
            cn, cd = xchg(1)
            cn.start()
            cd.start()
            combine_and_push(1)
            for ii in range(bq):
                cn, cd = xchg(ii)
                cn.wait_send()
                cd.wait_send()
                for r in range(3):
                    g = gath(r, ii)
                    g.wait_send()
                    g.wait_recv()

    grid_spec = pltpu.PrefetchScalarGridSpec(
        num_scalar_prefetch=1,
        grid=(bq, nc),
        in_specs=[
            pl.BlockSpec((1, q, h, d), lambda i, c, tr: (tr[0] * bq + i, 0, 0, 0)),
            pl.BlockSpec((1, kc, h, d), lambda i, c, tr: (tr[0] * bq + i, c, 0, 0)),
            pl.BlockSpec((1, kc, h, d), lambda i, c, tr: (tr[0] * bq + i, c, 0, 0)),
        ],
        out_specs=pl.BlockSpec((b, q, h, d), lambda i, c, tr: (0, 0, 0, 0)),
        scratch_shapes=[
            pltpu.VMEM((rows, cols), jnp.float32),
            pltpu.VMEM((bq, rows, d), jnp.float32),
            pltpu.VMEM((bq, rows, 1), jnp.float32),
            pltpu.VMEM((bq, rows, d), jnp.float32),
            pltpu.VMEM((bq, rows, 1), jnp.float32),
            pltpu.SemaphoreType.DMA((2, 2)),
            pltpu.SemaphoreType.DMA((2, 2)),
            pltpu.SemaphoreType.DMA((3, 2)),
            pltpu.SemaphoreType.DMA((3, 2)),
        ],
    )

    return pl.pallas_call(
        body,
        grid_spec=grid_spec,
        out_shape=jax.ShapeDtypeStruct((b, q, h, d), jnp.float32),
        compiler_params=pltpu.CompilerParams(collective_id=0),
    )(t_arr, Q, K, V)
